# baseline (speedup 1.0000x reference)
import jax
import jax.numpy as jnp
from jax import lax
from jax.experimental import pallas as pl
from jax.experimental.pallas import tpu as pltpu
from jax.experimental.pallas import tpu_sc as plsc

B = 8
H = W = 112
C = 96
N = H * W * C
M = 4 * N
NC = 2
NS = 16
LANES = 16
NCHUNK = 3
CHUNK = M // NCHUNK
NPASS = B * NCHUNK
HT = H // NS
ROWS_T = HT * C
PR = 24
P = PR * W
NPIECE = ROWS_T // PR
NW16 = W // LANES
NTZ = CHUNK // NS
ZB = 3136
NZP = NTZ // ZB


def _unpool_body(grad_hbm, arg_hbm, out_hbm, acc,
                 idx0, val0, idx1, val1, si0, sv0, si1, sv1, zer_v,
                 lsem0, lsem1, ssem0, ssem1, zsem):
    cid = lax.axis_index("c")
    sid = lax.axis_index("s")

    idx_s = (idx0, idx1)
    val_s = (val0, val1)
    si_s = (si0, si1)
    sv_s = (sv0, sv1)
    lsem = (lsem0, lsem1)
    ssem = (ssem0, ssem1)

    zeros16 = jnp.zeros((LANES,), jnp.float32)

    def fill_zer(i, _):
        zer_v[pl.ds(i * LANES, LANES)] = zeros16
        return _

    lax.fori_loop(0, ZB // LANES, fill_zer, None)

    def start_loads(b, p, slot):
        row0 = (b * H + HT * sid) * C + p * PR
        pltpu.make_async_copy(
            arg_hbm.at[pl.ds(row0, PR), :], idx_s[slot], lsem[slot]).start()
        pltpu.make_async_copy(
            grad_hbm.at[pl.ds(row0, PR), :], val_s[slot], lsem[slot]).start()

    def wait_loads(b, p, slot):
        row0 = (b * H + HT * sid) * C + p * PR
        pltpu.make_async_copy(
            arg_hbm.at[pl.ds(row0, PR), :], idx_s[slot], lsem[slot]).wait()
        pltpu.make_async_copy(
            grad_hbm.at[pl.ds(row0, PR), :], val_s[slot], lsem[slot]).wait()

    def pass_body(k, _):
        t = 2 * k + cid
        b = t // NCHUNK
        ch = t % NCHUNK
        lo = ch * CHUNK

        def zero_body(j, _):
            pltpu.make_async_copy(
                zer_v, acc.at[pl.ds(sid * NTZ + j * ZB, ZB)], zsem).start()
            return _

        lax.fori_loop(0, NZP, zero_body, None)

        def zero_wait(j, _):
            pltpu.make_async_copy(
                zer_v, acc.at[pl.ds(sid * NTZ + j * ZB, ZB)], zsem).wait()
            return _

        lax.fori_loop(0, NZP, zero_wait, None)
        plsc.subcore_barrier()

        start_loads(b, 0, 0)
        for p in range(NPIECE):
            slot = p % 2
            wait_loads(b, p, slot)

            def remap_body(r, _, slot=slot):
                for u in range(NW16):
                    sl = pl.ds(u * LANES, LANES)
                    so = pl.ds((r * NW16 + u) * LANES, LANES)
                    iv = idx_s[slot][r, sl]
                    vv = val_s[slot][r, sl]
                    m = (iv >= lo) & (iv < lo + CHUNK)
                    si_s[slot][so] = jnp.where(m, iv - lo, iv >> 2)
                    sv_s[slot][so] = jnp.where(m, vv, 0.0)
                return _

            lax.fori_loop(0, PR, remap_body, None)
            pltpu.make_async_copy(
                sv_s[slot], acc.at[si_s[slot]], ssem[slot]
            ).start(add=True)
            if p + 1 < NPIECE:
                if p >= 1:
                    pltpu.make_async_copy(
                        sv_s[1 - slot], acc.at[si_s[1 - slot]], ssem[1 - slot]
                    ).wait()
                start_loads(b, p + 1, 1 - slot)

        pltpu.make_async_copy(sv0, acc.at[si0], ssem0).wait()
        pltpu.make_async_copy(sv1, acc.at[si1], ssem1).wait()
        plsc.subcore_barrier()

        pltpu.sync_copy(
            acc.at[pl.ds(sid * NTZ, NTZ)],
            out_hbm.at[pl.ds(b * M + lo + sid * NTZ, NTZ)],
        )
        return _

    lax.fori_loop(0, NPASS // NC, pass_body, None)


@jax.jit
def _unpool(grad2, arg2):
    mesh = plsc.VectorSubcoreMesh(core_axis_name="c", subcore_axis_name="s")
    return pl.kernel(
        _unpool_body,
        out_type=jax.ShapeDtypeStruct((B * M,), jnp.float32),
        mesh=mesh,
        compiler_params=pltpu.CompilerParams(
            needs_layout_passes=False, use_tc_tiling_on_sc=True),
        scratch_types=[
            pltpu.VMEM_SHARED((CHUNK,), jnp.float32),
            pltpu.VMEM((PR, W), jnp.int32),
            pltpu.VMEM((PR, W), jnp.float32),
            pltpu.VMEM((PR, W), jnp.int32),
            pltpu.VMEM((PR, W), jnp.float32),
            pltpu.VMEM((P,), jnp.int32),
            pltpu.VMEM((P,), jnp.float32),
            pltpu.VMEM((P,), jnp.int32),
            pltpu.VMEM((P,), jnp.float32),
            pltpu.VMEM((ZB,), jnp.float32),
            pltpu.SemaphoreType.DMA,
            pltpu.SemaphoreType.DMA,
            pltpu.SemaphoreType.DMA,
            pltpu.SemaphoreType.DMA,
            pltpu.SemaphoreType.DMA,
        ],
    )(grad2, arg2)


def kernel(grad_out, inputs, argmax, batch_size):
    del inputs, batch_size
    grad2 = grad_out.transpose(0, 1, 3, 2).reshape(B * H * C, W)
    arg2 = argmax.transpose(0, 1, 3, 2).reshape(B * H * C, W).astype(jnp.int32)
    out_flat = _unpool(grad2, arg2)
    return out_flat.reshape(B, 2 * H, 2 * W, C)


# speedup vs baseline: 1.1972x; 1.1972x over previous
import jax
import jax.numpy as jnp
from jax import lax
from jax.experimental import pallas as pl
from jax.experimental.pallas import tpu as pltpu
from jax.experimental.pallas import tpu_sc as plsc

B = 8
H = W = 112
C = 96
N = H * W * C
M = 4 * N
NC = 2
NS = 16
LANES = 16
NCHUNK = 3
CHUNK = M // NCHUNK
NPASS = B * NCHUNK
NT = N // NS
P = 6272
NPIECE = NT // P
NTZ = CHUNK // NS
ZB = 3136
NZP = NTZ // ZB
UNROLL = 8


def _unpool_body(grad_hbm, arg_hbm, out_hbm, acc,
                 idx0, val0, idx1, val1, zer_v,
                 lsem0, lsem1, ssem0, ssem1, zsem):
    cid = lax.axis_index("c")
    sid = lax.axis_index("s")

    idx_s = (idx0, idx1)
    val_s = (val0, val1)
    lsem = (lsem0, lsem1)
    ssem = (ssem0, ssem1)

    zeros16 = jnp.zeros((LANES,), jnp.float32)

    def fill_zer(i, _):
        zer_v[pl.ds(i * LANES, LANES)] = zeros16
        return _

    lax.fori_loop(0, ZB // LANES, fill_zer, None)

    def start_loads(b, p, slot):
        base = b * N + sid * NT + p * P
        pltpu.make_async_copy(
            arg_hbm.at[pl.ds(base, P)], idx_s[slot], lsem[slot]).start()
        pltpu.make_async_copy(
            grad_hbm.at[pl.ds(base, P)], val_s[slot], lsem[slot]).start()

    def wait_loads(b, p, slot):
        base = b * N + sid * NT + p * P
        pltpu.make_async_copy(
            arg_hbm.at[pl.ds(base, P)], idx_s[slot], lsem[slot]).wait()
        pltpu.make_async_copy(
            grad_hbm.at[pl.ds(base, P)], val_s[slot], lsem[slot]).wait()

    def pass_body(k, _):
        t = 2 * k + cid
        b = t // NCHUNK
        ch = t % NCHUNK
        lo = ch * CHUNK

        def zero_body(j, _):
            pltpu.make_async_copy(
                zer_v, acc.at[pl.ds(sid * NTZ + j * ZB, ZB)], zsem).start()
            return _

        lax.fori_loop(0, NZP, zero_body, None)

        def zero_wait(j, _):
            pltpu.make_async_copy(
                zer_v, acc.at[pl.ds(sid * NTZ + j * ZB, ZB)], zsem).wait()
            return _

        lax.fori_loop(0, NZP, zero_wait, None)
        plsc.subcore_barrier()

        start_loads(b, 0, 0)
        for p in range(NPIECE):
            slot = p % 2
            wait_loads(b, p, slot)

            def remap_body(i, _, slot=slot):
                for u in range(UNROLL):
                    sl = pl.ds(i * (LANES * UNROLL) + u * LANES, LANES)
                    iv = idx_s[slot][sl]
                    vv = val_s[slot][sl]
                    m = (iv >= lo) & (iv < lo + CHUNK)
                    idx_s[slot][sl] = jnp.where(m, iv - lo, iv >> 2)
                    val_s[slot][sl] = jnp.where(m, vv, 0.0)
                return _

            lax.fori_loop(0, P // (LANES * UNROLL), remap_body, None)
            pltpu.make_async_copy(
                val_s[slot], acc.at[idx_s[slot]], ssem[slot]
            ).start(add=True)
            if p + 1 < NPIECE:
                if p >= 1:
                    pltpu.make_async_copy(
                        val_s[1 - slot], acc.at[idx_s[1 - slot]], ssem[1 - slot]
                    ).wait()
                start_loads(b, p + 1, 1 - slot)

        pltpu.make_async_copy(val0, acc.at[idx0], ssem0).wait()
        pltpu.make_async_copy(val1, acc.at[idx1], ssem1).wait()
        plsc.subcore_barrier()

        pltpu.sync_copy(
            acc.at[pl.ds(sid * NTZ, NTZ)],
            out_hbm.at[pl.ds(b * M + lo + sid * NTZ, NTZ)],
        )
        return _

    lax.fori_loop(0, NPASS // NC, pass_body, None)


@jax.jit
def _unpool(grad_flat, arg_flat):
    mesh = plsc.VectorSubcoreMesh(core_axis_name="c", subcore_axis_name="s")
    return pl.kernel(
        _unpool_body,
        out_type=jax.ShapeDtypeStruct((B * M,), jnp.float32),
        mesh=mesh,
        compiler_params=pltpu.CompilerParams(needs_layout_passes=False),
        scratch_types=[
            pltpu.VMEM_SHARED((CHUNK,), jnp.float32),
            pltpu.VMEM((P,), jnp.int32),
            pltpu.VMEM((P,), jnp.float32),
            pltpu.VMEM((P,), jnp.int32),
            pltpu.VMEM((P,), jnp.float32),
            pltpu.VMEM((ZB,), jnp.float32),
            pltpu.SemaphoreType.DMA,
            pltpu.SemaphoreType.DMA,
            pltpu.SemaphoreType.DMA,
            pltpu.SemaphoreType.DMA,
            pltpu.SemaphoreType.DMA,
        ],
    )(grad_flat, arg_flat)


def kernel(grad_out, inputs, argmax, batch_size):
    del inputs, batch_size
    grad_flat = grad_out.reshape(B * N)
    arg_flat = argmax.reshape(B * N).astype(jnp.int32)
    out_flat = _unpool(grad_flat, arg_flat)
    return out_flat.reshape(B, 2 * H, 2 * W, C)


# speedup vs baseline: 1.1995x; 1.0020x over previous
import jax
import jax.numpy as jnp
from jax import lax
from jax.experimental import pallas as pl
from jax.experimental.pallas import tpu as pltpu
from jax.experimental.pallas import tpu_sc as plsc

B = 8
H = W = 112
C = 96
N = H * W * C
M = 4 * N
NC = 2
NS = 16
LANES = 16
NCHUNK = 3
CHUNK = M // NCHUNK
NPASS = B * NCHUNK
NT = N // NS
P = 6272
NPIECE = NT // P
NTZ = CHUNK // NS
ZB = 3136
NZP = NTZ // ZB
UNROLL = 8


def _unpool_body(grad_hbm, arg_hbm, out_hbm, acc,
                 idx0, val0, idx1, val1, zer_v,
                 lsem0, lsem1, ssem0, ssem1, zsem):
    cid = lax.axis_index("c")
    sid = lax.axis_index("s")

    idx_s = (idx0, idx1)
    val_s = (val0, val1)
    lsem = (lsem0, lsem1)
    ssem = (ssem0, ssem1)

    zeros16 = jnp.zeros((LANES,), jnp.float32)

    def fill_zer(i, _):
        zer_v[pl.ds(i * LANES, LANES)] = zeros16
        return _

    lax.fori_loop(0, ZB // LANES, fill_zer, None)

    def start_loads(b, p, slot):
        base = b * N + sid * NT + p * P
        pltpu.make_async_copy(
            arg_hbm.at[pl.ds(base, P)], idx_s[slot], lsem[slot]).start()
        pltpu.make_async_copy(
            grad_hbm.at[pl.ds(base, P)], val_s[slot], lsem[slot]).start()

    def wait_loads(b, p, slot):
        base = b * N + sid * NT + p * P
        pltpu.make_async_copy(
            arg_hbm.at[pl.ds(base, P)], idx_s[slot], lsem[slot]).wait()
        pltpu.make_async_copy(
            grad_hbm.at[pl.ds(base, P)], val_s[slot], lsem[slot]).wait()

    def pass_body(k, _):
        t = 2 * k + cid
        b = t // NCHUNK
        ch = t % NCHUNK
        lo = ch * CHUNK

        start_loads(b, 0, 0)
        start_loads(b, 1, 1)

        def zero_body(j, _):
            pltpu.make_async_copy(
                zer_v, acc.at[pl.ds(sid * NTZ + j * ZB, ZB)], zsem).start()
            return _

        lax.fori_loop(0, NZP, zero_body, None)

        def zero_wait(j, _):
            pltpu.make_async_copy(
                zer_v, acc.at[pl.ds(sid * NTZ + j * ZB, ZB)], zsem).wait()
            return _

        lax.fori_loop(0, NZP, zero_wait, None)
        plsc.subcore_barrier()

        for p in range(NPIECE):
            slot = p % 2
            wait_loads(b, p, slot)

            def remap_body(i, _, slot=slot):
                for u in range(UNROLL):
                    sl = pl.ds(i * (LANES * UNROLL) + u * LANES, LANES)
                    iv = idx_s[slot][sl]
                    vv = val_s[slot][sl]
                    m = (iv >= lo) & (iv < lo + CHUNK)
                    idx_s[slot][sl] = jnp.where(m, iv - lo, iv >> 2)
                    val_s[slot][sl] = jnp.where(m, vv, 0.0)
                return _

            lax.fori_loop(0, P // (LANES * UNROLL), remap_body, None)
            pltpu.make_async_copy(
                val_s[slot], acc.at[idx_s[slot]], ssem[slot]
            ).start(add=True)
            if p >= 1 and p + 1 < NPIECE:
                pltpu.make_async_copy(
                    val_s[1 - slot], acc.at[idx_s[1 - slot]], ssem[1 - slot]
                ).wait()
                start_loads(b, p + 1, 1 - slot)

        pltpu.make_async_copy(val0, acc.at[idx0], ssem0).wait()
        pltpu.make_async_copy(val1, acc.at[idx1], ssem1).wait()
        plsc.subcore_barrier()

        pltpu.sync_copy(
            acc.at[pl.ds(sid * NTZ, NTZ)],
            out_hbm.at[pl.ds(b * M + lo + sid * NTZ, NTZ)],
        )
        return _

    lax.fori_loop(0, NPASS // NC, pass_body, None)


@jax.jit
def _unpool(grad_flat, arg_flat):
    mesh = plsc.VectorSubcoreMesh(core_axis_name="c", subcore_axis_name="s")
    return pl.kernel(
        _unpool_body,
        out_type=jax.ShapeDtypeStruct((B * M,), jnp.float32),
        mesh=mesh,
        compiler_params=pltpu.CompilerParams(needs_layout_passes=False),
        scratch_types=[
            pltpu.VMEM_SHARED((CHUNK,), jnp.float32),
            pltpu.VMEM((P,), jnp.int32),
            pltpu.VMEM((P,), jnp.float32),
            pltpu.VMEM((P,), jnp.int32),
            pltpu.VMEM((P,), jnp.float32),
            pltpu.VMEM((ZB,), jnp.float32),
            pltpu.SemaphoreType.DMA,
            pltpu.SemaphoreType.DMA,
            pltpu.SemaphoreType.DMA,
            pltpu.SemaphoreType.DMA,
            pltpu.SemaphoreType.DMA,
        ],
    )(grad_flat, arg_flat)


def kernel(grad_out, inputs, argmax, batch_size):
    del inputs, batch_size
    grad_flat = grad_out.reshape(B * N)
    arg_flat = argmax.reshape(B * N).astype(jnp.int32)
    out_flat = _unpool(grad_flat, arg_flat)
    return out_flat.reshape(B, 2 * H, 2 * W, C)
